# baseline (speedup 1.0000x reference)
import functools

import jax
import jax.numpy as jnp
from jax import lax
from jax.experimental import pallas as pl
from jax.experimental.pallas import tpu as pltpu
from jax.experimental.pallas import tpu_sc as plsc

N = 10000
E = 320000
D = 128

NP = 10240
EP = 327680
EPR = EP // 128
NW = 32
RPT = EPR // NW
NPT = NP // 16
BM = 2048

_mesh = plsc.VectorSubcoreMesh(core_axis_name="c", subcore_axis_name="s")


@functools.partial(
    pl.kernel,
    out_type=jax.ShapeDtypeStruct((2, NP, 16), jnp.float32),
    mesh=_mesh,
    scratch_types=[
        pltpu.VMEM((RPT, 128), jnp.int32),
        pltpu.VMEM((128, 16), jnp.float32),
        pltpu.VMEM_SHARED((NP, 16), jnp.float32),
    ],
)
def _deg_kernel(dstp, zdeg, out, di_v, ones_v, acc_sp):
    c = lax.axis_index("c")
    s = lax.axis_index("s")

    def fill_ones(i, carry):
        ones_v[i] = jnp.ones((16,), jnp.float32)
        return carry

    lax.fori_loop(0, 128, fill_ones, 0)

    base = s * NPT
    pltpu.sync_copy(zdeg.at[pl.ds(base, NPT)], acc_sp.at[pl.ds(base, NPT)])
    plsc.subcore_barrier()

    row0 = c * (EPR // 2) + s * RPT
    pltpu.sync_copy(dstp.at[pl.ds(row0, RPT)], di_v)

    def step(j, carry):
        pltpu.sync_copy(ones_v, acc_sp.at[di_v.at[j]], add=True)
        return carry

    lax.fori_loop(0, RPT, step, 0)
    plsc.subcore_barrier()
    pltpu.sync_copy(acc_sp.at[pl.ds(base, NPT)], out.at[c, pl.ds(base, NPT)])


IB = 40
NS = 1
SUB = 128 // NS


@functools.partial(
    pl.kernel,
    out_type=jax.ShapeDtypeStruct((2, NP, D), jnp.float32),
    mesh=_mesh,
    scratch_types=[
        pltpu.VMEM((IB, 128), jnp.int32),
        pltpu.VMEM((IB, 128), jnp.int32),
        pltpu.VMEM((128, D), jnp.float32),
        pltpu.VMEM((128, D), jnp.float32),
        [pltpu.SemaphoreType.DMA] * (2 * NS),
        pltpu.VMEM_SHARED((NP, D), jnp.float32),
    ],
)
def _mp_kernel(g, srcp, dstp, znode, out, si_v, di_v, rows_a, rows_b,
               sems, acc_sp):
    c = lax.axis_index("c")
    s = lax.axis_index("s")

    base = s * NPT
    pltpu.sync_copy(znode.at[pl.ds(base, NPT)], acc_sp.at[pl.ds(base, NPT)])
    plsc.subcore_barrier()

    row0 = c * (EPR // 2) + s * RPT

    def step_pair(j, carry2):
        cps = []
        for p, buf in ((0, rows_a), (1, rows_b)):
            for k in range(NS):
                cps.append(pltpu.async_copy(
                    g.at[si_v.at[j + p, pl.ds(k * SUB, SUB)]],
                    buf.at[pl.ds(k * SUB, SUB)],
                    sems[p * NS + k]))
        for cp in cps:
            cp.wait()
        pltpu.sync_copy(rows_a, acc_sp.at[di_v.at[j]], add=True)
        pltpu.sync_copy(rows_b, acc_sp.at[di_v.at[j + 1]], add=True)
        return carry2

    def outer(q, carry):
        pltpu.sync_copy(srcp.at[pl.ds(row0 + q * IB, IB)], si_v)
        pltpu.sync_copy(dstp.at[pl.ds(row0 + q * IB, IB)], di_v)
        lax.fori_loop(0, IB // 2, lambda t, c2: step_pair(2 * t, c2), 0)
        return carry

    lax.fori_loop(0, RPT // IB, outer, 0)
    plsc.subcore_barrier()
    pltpu.sync_copy(acc_sp.at[pl.ds(base, NPT)], out.at[c, pl.ds(base, NPT)])


def _mm_body(x_ref, w_ref, o_ref):
    o_ref[...] = jnp.dot(x_ref[...], w_ref[...],
                         preferred_element_type=jnp.float32,
                         precision=lax.Precision.HIGHEST)


_mm_call = pl.pallas_call(
    _mm_body,
    grid=(NP // BM,),
    in_specs=[
        pl.BlockSpec((BM, D), lambda i: (i, 0)),
        pl.BlockSpec((D, D), lambda i: (0, 0)),
    ],
    out_specs=pl.BlockSpec((BM, D), lambda i: (i, 0)),
    out_shape=jax.ShapeDtypeStruct((NP, D), jnp.float32),
)


def _scale_body(h_ref, d0_ref, d1_ref, g_ref, gn_ref):
    deg = d0_ref[0, :, :1] + d1_ref[0, :, :1] + 1.0
    norm = lax.rsqrt(deg)
    gv = h_ref[...] * norm
    g_ref[...] = gv
    gn_ref[...] = gv * norm


_scale_call = pl.pallas_call(
    _scale_body,
    grid=(NP // BM,),
    in_specs=[
        pl.BlockSpec((BM, D), lambda i: (i, 0)),
        pl.BlockSpec((1, BM, 16), lambda i: (0, i, 0)),
        pl.BlockSpec((1, BM, 16), lambda i: (1, i, 0)),
    ],
    out_specs=[
        pl.BlockSpec((BM, D), lambda i: (i, 0)),
        pl.BlockSpec((BM, D), lambda i: (i, 0)),
    ],
    out_shape=[
        jax.ShapeDtypeStruct((NP, D), jnp.float32),
        jax.ShapeDtypeStruct((NP, D), jnp.float32),
    ],
)


def _out_body(a0_ref, a1_ref, d0_ref, d1_ref, gn_ref, b_ref, o_ref):
    deg = d0_ref[0, :, :1] + d1_ref[0, :, :1] + 1.0
    norm = lax.rsqrt(deg)
    acc = a0_ref[0] + a1_ref[0]
    o_ref[...] = acc * norm + gn_ref[...] + b_ref[...]


_out_call = pl.pallas_call(
    _out_body,
    grid=(NP // BM,),
    in_specs=[
        pl.BlockSpec((1, BM, D), lambda i: (0, i, 0)),
        pl.BlockSpec((1, BM, D), lambda i: (1, i, 0)),
        pl.BlockSpec((1, BM, 16), lambda i: (0, i, 0)),
        pl.BlockSpec((1, BM, 16), lambda i: (1, i, 0)),
        pl.BlockSpec((BM, D), lambda i: (i, 0)),
        pl.BlockSpec((D,), lambda i: (0,)),
    ],
    out_specs=pl.BlockSpec((BM, D), lambda i: (i, 0)),
    out_shape=jax.ShapeDtypeStruct((NP, D), jnp.float32),
)


def kernel(x, edge_index, weight, bias):
    src = edge_index[0]
    dst = edge_index[1]
    pad = EP - E
    src_p = jnp.concatenate(
        [src, jnp.zeros((pad,), jnp.int32)]).reshape(EPR, 128)
    dst_p = jnp.concatenate(
        [dst, jnp.full((pad,), N, jnp.int32)]).reshape(EPR, 128)
    x_p = jnp.pad(x, ((0, NP - N), (0, 0)))

    zdeg = jnp.zeros((NP, 16), jnp.float32)
    znode = jnp.zeros((NP, D), jnp.float32)

    degp = _deg_kernel(dst_p, zdeg)
    h = _mm_call(x_p, weight)
    g, gn = _scale_call(h, degp, degp)
    accp = _mp_kernel(g, src_p, dst_p, znode)
    out = _out_call(accp, accp, degp, degp, gn, bias)
    return out[:N]


# speedup vs baseline: 1.1959x; 1.1959x over previous
import functools

import jax
import jax.numpy as jnp
from jax import lax
from jax.experimental import pallas as pl
from jax.experimental.pallas import tpu as pltpu
from jax.experimental.pallas import tpu_sc as plsc

N = 10000
E = 320000
D = 128

NP = 10240
EP = 327680
EPR = EP // 128
NW = 32
RPT = EPR // NW
NPT = NP // 16
BM = 2048

_mesh = plsc.VectorSubcoreMesh(core_axis_name="c", subcore_axis_name="s")


@functools.partial(
    pl.kernel,
    out_type=jax.ShapeDtypeStruct((2, NP, 16), jnp.float32),
    mesh=_mesh,
    scratch_types=[
        pltpu.VMEM((RPT, 128), jnp.int32),
        pltpu.VMEM((128, 16), jnp.float32),
        pltpu.VMEM_SHARED((NP, 16), jnp.float32),
    ],
)
def _deg_kernel(dstp, zdeg, out, di_v, ones_v, acc_sp):
    c = lax.axis_index("c")
    s = lax.axis_index("s")

    def fill_ones(i, carry):
        ones_v[i] = jnp.ones((16,), jnp.float32)
        return carry

    lax.fori_loop(0, 128, fill_ones, 0)

    base = s * NPT
    pltpu.sync_copy(zdeg.at[pl.ds(base, NPT)], acc_sp.at[pl.ds(base, NPT)])
    plsc.subcore_barrier()

    row0 = c * (EPR // 2) + s * RPT
    pltpu.sync_copy(dstp.at[pl.ds(row0, RPT)], di_v)

    def step(j, carry):
        pltpu.sync_copy(ones_v, acc_sp.at[di_v.at[j]], add=True)
        return carry

    lax.fori_loop(0, RPT, step, 0)
    plsc.subcore_barrier()
    pltpu.sync_copy(acc_sp.at[pl.ds(base, NPT)], out.at[c, pl.ds(base, NPT)])


NT = 10000
HALF = NP // 2
AR = 5248
ART = AR // 16
EC = EP // 32
CPT = EC // 16
IB = 8


@functools.partial(
    pl.kernel,
    out_type=jax.ShapeDtypeStruct((2, AR, D), jnp.float32),
    mesh=_mesh,
    scratch_types=[
        pltpu.VMEM((IB, 32), jnp.int32),
        pltpu.VMEM((IB, 32), jnp.int32),
        pltpu.VMEM((IB, 32), jnp.int32),
        pltpu.VMEM((32, D), jnp.float32),
        pltpu.SemaphoreType.DMA,
        pltpu.VMEM_SHARED((NT, D), jnp.float32),
        pltpu.VMEM_SHARED((AR, D), jnp.float32),
    ],
)
def _mp_kernel(g, srcp, dstp, znode, out, si_v, di_v, dl_v, rows_v, sem,
               tab_sp, acc_sp):
    c = lax.axis_index("c")
    s = lax.axis_index("s")

    abase = s * ART
    pltpu.sync_copy(znode.at[pl.ds(0, ART)], acc_sp.at[pl.ds(abase, ART)])

    @pl.when(s < 15)
    def _stage():
        pltpu.sync_copy(g.at[pl.ds(s * 632, 632)],
                        tab_sp.at[pl.ds(s * 632, 632)])

    @pl.when(s == 15)
    def _stage_tail():
        pltpu.sync_copy(g.at[pl.ds(15 * 632, NT - 15 * 632)],
                        tab_sp.at[pl.ds(15 * 632, NT - 15 * 632)])

    plsc.subcore_barrier()

    row0 = s * CPT
    dbase = c * HALF

    def outer(q, carry):
        r = row0 + q * IB
        pltpu.sync_copy(srcp.at[pl.ds(r, IB)], si_v)
        pltpu.sync_copy(dstp.at[pl.ds(r, IB)], di_v)
        for j in range(IB):
            for h in range(2):
                v = di_v[j, pl.ds(h * 16, 16)]
                inr = (v >= dbase) & (v < dbase + HALF)
                dl_v[j, pl.ds(h * 16, 16)] = jnp.where(
                    inr, v - dbase, HALF + 7)

        def step(j, carry2):
            pltpu.async_copy(tab_sp.at[si_v.at[j]], rows_v, sem).wait()
            pltpu.sync_copy(rows_v, acc_sp.at[dl_v.at[j]], add=True)
            return carry2

        lax.fori_loop(0, IB, step, 0)
        return carry

    lax.fori_loop(0, CPT // IB, outer, 0)
    plsc.subcore_barrier()
    pltpu.sync_copy(acc_sp.at[pl.ds(abase, ART)], out.at[c, pl.ds(abase, ART)])


def _mm_body(x_ref, w_ref, o_ref):
    o_ref[...] = jnp.dot(x_ref[...], w_ref[...],
                         preferred_element_type=jnp.float32,
                         precision=lax.Precision.HIGHEST)


_mm_call = pl.pallas_call(
    _mm_body,
    grid=(NP // BM,),
    in_specs=[
        pl.BlockSpec((BM, D), lambda i: (i, 0)),
        pl.BlockSpec((D, D), lambda i: (0, 0)),
    ],
    out_specs=pl.BlockSpec((BM, D), lambda i: (i, 0)),
    out_shape=jax.ShapeDtypeStruct((NP, D), jnp.float32),
)


def _scale_body(h_ref, d0_ref, d1_ref, g_ref, gn_ref):
    deg = d0_ref[0, :, :1] + d1_ref[0, :, :1] + 1.0
    norm = lax.rsqrt(deg)
    gv = h_ref[...] * norm
    g_ref[...] = gv
    gn_ref[...] = gv * norm


_scale_call = pl.pallas_call(
    _scale_body,
    grid=(NP // BM,),
    in_specs=[
        pl.BlockSpec((BM, D), lambda i: (i, 0)),
        pl.BlockSpec((1, BM, 16), lambda i: (0, i, 0)),
        pl.BlockSpec((1, BM, 16), lambda i: (1, i, 0)),
    ],
    out_specs=[
        pl.BlockSpec((BM, D), lambda i: (i, 0)),
        pl.BlockSpec((BM, D), lambda i: (i, 0)),
    ],
    out_shape=[
        jax.ShapeDtypeStruct((NP, D), jnp.float32),
        jax.ShapeDtypeStruct((NP, D), jnp.float32),
    ],
)


BO = 1024


def _out_body(a_ref, d0_ref, d1_ref, gn_ref, b_ref, o_ref):
    deg = d0_ref[0, :, :1] + d1_ref[0, :, :1] + 1.0
    norm = lax.rsqrt(deg)
    o_ref[...] = a_ref[0] * norm + gn_ref[...] + b_ref[...]


_out_call = pl.pallas_call(
    _out_body,
    grid=(NP // BO,),
    in_specs=[
        pl.BlockSpec((1, BO, D), lambda i: (i // 5, i % 5, 0)),
        pl.BlockSpec((1, BO, 16), lambda i: (0, i, 0)),
        pl.BlockSpec((1, BO, 16), lambda i: (1, i, 0)),
        pl.BlockSpec((BO, D), lambda i: (i, 0)),
        pl.BlockSpec((D,), lambda i: (0,)),
    ],
    out_specs=pl.BlockSpec((BO, D), lambda i: (i, 0)),
    out_shape=jax.ShapeDtypeStruct((NP, D), jnp.float32),
)


def kernel(x, edge_index, weight, bias):
    src = edge_index[0]
    dst = edge_index[1]
    pad = EP - E
    src_p = jnp.concatenate(
        [src, jnp.zeros((pad,), jnp.int32)]).reshape(EC, 32)
    dst_p = jnp.concatenate(
        [dst, jnp.full((pad,), N, jnp.int32)]).reshape(EC, 32)
    dst_p128 = jnp.reshape(dst_p, (EPR, 128))
    x_p = jnp.pad(x, ((0, NP - N), (0, 0)))

    zdeg = jnp.zeros((NP, 16), jnp.float32)
    znode = jnp.zeros((NP, D), jnp.float32)

    degp = _deg_kernel(dst_p128, zdeg)
    h = _mm_call(x_p, weight)
    g, gn = _scale_call(h, degp, degp)
    accp = _mp_kernel(g, src_p, dst_p, znode)
    out = _out_call(accp, degp, degp, gn, bias)
    return out[:N]
